# baseline (speedup 1.0000x reference)
import functools

import jax
import jax.numpy as jnp
from jax import lax
from jax.experimental import pallas as pl
from jax.experimental.pallas import tpu as pltpu
from jax.experimental.pallas import tpu_sc as plsc

_D = 128
_NW = 32
_CHUNK = 128
_NBUF = 5
_GAHEAD = 2
_OAHEAD = 3


@functools.partial(jax.jit, static_argnames=("n_chunks",))
def _gather_rows(idx, table, n_chunks):
    mesh = plsc.VectorSubcoreMesh(core_axis_name="c", subcore_axis_name="s")

    @functools.partial(
        pl.kernel,
        mesh=mesh,
        out_type=jax.ShapeDtypeStruct((_NW * n_chunks * _CHUNK, _D), jnp.float32),
        scratch_types=[
            pltpu.VMEM((n_chunks, _CHUNK), jnp.int32),
            pltpu.VMEM((_NBUF, _CHUNK, _D), jnp.float32),
            pltpu.SemaphoreType.DMA,
            pltpu.SemaphoreType.DMA,
        ],
    )
    def k(idx_hbm, table_hbm, out_hbm, idx_v, rows_v, gsem, osem):
        wid = lax.axis_index("s") * 2 + lax.axis_index("c")
        base = wid * (n_chunks * _CHUNK)
        pltpu.sync_copy(idx_hbm.at[wid], idx_v)

        def gather_start(j, b):
            pltpu.make_async_copy(
                table_hbm.at[idx_v.at[j]], rows_v.at[b], gsem
            ).start()

        def gather_wait(j, b):
            pltpu.make_async_copy(
                table_hbm.at[idx_v.at[j]], rows_v.at[b], gsem
            ).wait()

        def out_start(j, b):
            pltpu.make_async_copy(
                rows_v.at[b], out_hbm.at[pl.ds(base + j * _CHUNK, _CHUNK)], osem
            ).start()

        def out_wait(j, b):
            pltpu.make_async_copy(
                rows_v.at[b], out_hbm.at[pl.ds(base + j * _CHUNK, _CHUNK)], osem
            ).wait()

        for j in range(_GAHEAD):
            gather_start(j, j % _NBUF)

        def body(i, _):
            jj = i * _NBUF
            for b in range(_NBUF):
                j = jj + b
                @pl.when(j >= _OAHEAD)
                def _():
                    out_wait(j - _OAHEAD, (b - _OAHEAD) % _NBUF)

                gather_wait(j, b)
                out_start(j, b)

                @pl.when(j + _GAHEAD < n_chunks)
                def _():
                    gather_start(j + _GAHEAD, (b + _GAHEAD) % _NBUF)
            return 0

        lax.fori_loop(0, n_chunks // _NBUF, body, 0)
        for j in range(n_chunks - _OAHEAD, n_chunks):
            out_wait(j, j % _NBUF)

    return k(idx, table)


def kernel(x, table):
    n_total = x.shape[0] * x.shape[1]
    per_w = n_total // _NW
    n_chunks = per_w // _CHUNK
    idx = x.reshape(_NW, n_chunks, _CHUNK).astype(jnp.int32)
    out = _gather_rows(idx, table, n_chunks)
    return out.reshape(x.shape[0], x.shape[1], _D)


# speedup vs baseline: 1.0065x; 1.0065x over previous
import functools

import jax
import jax.numpy as jnp
from jax import lax
from jax.experimental import pallas as pl
from jax.experimental.pallas import tpu as pltpu
from jax.experimental.pallas import tpu_sc as plsc

_D = 128
_NW = 32
_CHUNK = 128
_NBUF = 5
_GAHEAD = 2
_OAHEAD = 3


@functools.partial(jax.jit, static_argnames=("n_chunks",))
def _gather_rows(idx, table, n_chunks):
    mesh = plsc.VectorSubcoreMesh(core_axis_name="c", subcore_axis_name="s")

    @functools.partial(
        pl.kernel,
        mesh=mesh,
        out_type=jax.ShapeDtypeStruct((_NW * n_chunks * _CHUNK, _D), jnp.float32),
        scratch_types=[
            pltpu.VMEM((n_chunks, _CHUNK), jnp.int32),
            pltpu.VMEM((_NBUF, _CHUNK, _D), jnp.float32),
            pltpu.SemaphoreType.DMA,
            pltpu.SemaphoreType.DMA,
        ],
    )
    def k(idx_hbm, table_hbm, out_hbm, idx_v, rows_v, gsem, osem):
        wid = lax.axis_index("s") * 2 + lax.axis_index("c")
        base = wid * (n_chunks * _CHUNK)
        pltpu.sync_copy(idx_hbm.at[wid], idx_v)

        def gather_start(j, b):
            pltpu.make_async_copy(
                table_hbm.at[idx_v.at[j]], rows_v.at[b], gsem
            ).start()

        def gather_wait(j, b):
            pltpu.make_async_copy(
                table_hbm.at[idx_v.at[j]], rows_v.at[b], gsem
            ).wait()

        def out_start(j, b):
            pltpu.make_async_copy(
                rows_v.at[b], out_hbm.at[pl.ds(base + j * _CHUNK, _CHUNK)], osem
            ).start()

        def out_wait(j, b):
            pltpu.make_async_copy(
                rows_v.at[b], out_hbm.at[pl.ds(base + j * _CHUNK, _CHUNK)], osem
            ).wait()

        for j in range(_GAHEAD):
            gather_start(j, j % _NBUF)

        def body(i, _):
            jj = i * _NBUF
            for b in range(_NBUF):
                j = jj + b
                @pl.when(j >= _OAHEAD)
                def _():
                    out_wait(j - _OAHEAD, (b - _OAHEAD) % _NBUF)

                @pl.when(j + _GAHEAD < n_chunks)
                def _():
                    gather_start(j + _GAHEAD, (b + _GAHEAD) % _NBUF)

                gather_wait(j, b)
                out_start(j, b)
            return 0

        lax.fori_loop(0, n_chunks // _NBUF, body, 0)
        for j in range(n_chunks - _OAHEAD, n_chunks):
            out_wait(j, j % _NBUF)

    return k(idx, table)


def kernel(x, table):
    n_total = x.shape[0] * x.shape[1]
    per_w = n_total // _NW
    n_chunks = per_w // _CHUNK
    idx = x.reshape(_NW, n_chunks, _CHUNK).astype(jnp.int32)
    out = _gather_rows(idx, table, n_chunks)
    return out.reshape(x.shape[0], x.shape[1], _D)
